# baseline (speedup 1.0000x reference)
import functools

import numpy as np
import jax
import jax.numpy as jnp
from jax import lax
from jax.experimental import pallas as pl
from jax.experimental.pallas import tpu as pltpu
from jax.experimental.pallas import tpu_sc as plsc

N, E, D, H, DK, R, DFF = 10000, 320000, 128, 8, 16, 100, 512
INV_SCALE = 1.0 / 4.0
NC, NS = 2, 16
NW = NC * NS
C = 32
TOT_CH = E // C
NCH_SMALL = TOT_CH // NW
WBIG = TOT_CH - NCH_SMALL * NW
NCH_BIG = NCH_SMALL + 1
ZC = 16
ZCH = N // ZC
N8 = 1280
Z8CH = N8 // ZC


BRQ = 1000


def _qkv_body(x_ref, w_ref, b_ref, q_ref, kv_ref):
    acc = jnp.dot(x_ref[...], w_ref[...], preferred_element_type=jnp.float32)
    acc = acc + b_ref[...]
    q_ref[...] = acc[:, :D]
    kv_ref[...] = acc[:, D:]


_qkv_call = pl.pallas_call(
    _qkv_body,
    grid=(N // BRQ,),
    in_specs=[
        pl.BlockSpec((BRQ, D), lambda i: (i, 0)),
        pl.BlockSpec((D, 3 * D), lambda i: (0, 0)),
        pl.BlockSpec((1, 3 * D), lambda i: (0, 0)),
    ],
    out_specs=[
        pl.BlockSpec((BRQ, D), lambda i: (i, 0)),
        pl.BlockSpec((BRQ, 2 * D), lambda i: (i, 0)),
    ],
    out_shape=[
        jax.ShapeDtypeStruct((N, D), jnp.float32),
        jax.ShapeDtypeStruct((N, 2 * D), jnp.float32),
    ],
)


def _edge_body(q_hbm, kv_hbm, rel_hbm, eidx_hbm,
               wv_hbm, zp_hbm,
               idx_v, dst_v, zidx_v, kv_v, q_v, msg_v, zmsg_v, rel_v,
               acc_sh, zacc_sh, sem_kv, sem_q):
    cid = lax.axis_index("c")
    sid = lax.axis_index("s")
    wid = sid * NC + cid

    zeros16 = jnp.zeros((16,), jnp.float32)

    def zrow(i, _):
        for j in range(D // 16):
            msg_v[i, pl.ds(j * 16, 16)] = zeros16
            zmsg_v[i, pl.ds(j * 16, 16)] = zeros16
        return 0
    lax.fori_loop(0, C, zrow, 0)

    pltpu.sync_copy(rel_hbm, rel_v)

    for j in range((ZCH + Z8CH + NS - 1) // NS):
        ch = sid + j * NS
        @pl.when(ch < ZCH)
        def _zero_acc():
            pltpu.sync_copy(msg_v.at[pl.ds(0, ZC)],
                            acc_sh.at[pl.ds(ch * ZC, ZC)])
        @pl.when(jnp.logical_and(ch >= ZCH, ch < ZCH + Z8CH))
        def _zero_zacc():
            pltpu.sync_copy(msg_v.at[pl.ds(0, ZC)],
                            zacc_sh.at[pl.ds((ch - ZCH) * ZC, ZC)])

    plsc.subcore_barrier()

    lane = lax.iota(jnp.int32, 16)
    perms = [jnp.bitwise_xor(lane, m)[:, None] for m in (8, 4, 2, 1)]
    bperms = [(lane * 0 + h)[:, None] for h in range(H)]
    dnums = lax.GatherDimensionNumbers(
        offset_dims=(), collapsed_slice_dims=(0,), start_index_map=(0,))

    nch = jnp.where(wid < WBIG, NCH_BIG, NCH_SMALL)
    ch0 = jnp.where(wid < WBIG, wid * NCH_BIG,
                    WBIG * NCH_BIG + (wid - WBIG) * NCH_SMALL)

    def chunk(i, _):
        pltpu.sync_copy(eidx_hbm.at[ch0 + i], idx_v)
        for t in range(C // 16):
            dvec16 = idx_v[pl.ds(C + t * 16, 16)]
            dst_v[pl.ds(t * 16, 16)] = dvec16
            zidx_v[pl.ds(t * 16, 16)] = lax.shift_right_logical(dvec16, 3)
        cp_kv = pltpu.async_copy(kv_hbm.at[idx_v.at[pl.ds(0, C)]], kv_v,
                                 sem_kv)
        cp_q = pltpu.async_copy(q_hbm.at[idx_v.at[pl.ds(C, C)]], q_v, sem_q)
        cp_kv.wait()
        cp_q.wait()

        def group(g):
            dvec = dst_v[pl.ds(g * 16, 16)]
            fvec = idx_v[pl.ds(2 * C + g * 16, 16)]
            for l in range(16):
                e = g * 16 + l
                f_s = fvec[l]
                d_s = dvec[l]
                ev = rel_v[pl.ds(f_s * DK, DK)]
                score_vec = zeros16
                for h in range(H):
                    kvec = kv_v[e, pl.ds(h * DK, DK)]
                    qvec = q_v[e, pl.ds(h * DK, DK)]
                    t = (kvec + ev) * qvec
                    for p in perms:
                        t = t + lax.gather(
                            t, p, dnums, (1,),
                            mode=lax.GatherScatterMode.PROMISE_IN_BOUNDS)
                    score_vec = jnp.where(lane == h, t, score_vec)
                escore = jnp.exp(jnp.clip(score_vec * INV_SCALE, -10.0, 10.0))
                for h in range(H):
                    sv = lax.gather(
                        escore, bperms[h], dnums, (1,),
                        mode=lax.GatherScatterMode.PROMISE_IN_BOUNDS)
                    vvec = kv_v[e, pl.ds(D + h * DK, DK)]
                    msg_v[e, pl.ds(h * DK, DK)] = (vvec + ev) * sv
                zrow_vec = jnp.where(lane < H, escore, zeros16)
                blk = jnp.bitwise_and(d_s, 7)
                for j in range(8):
                    val = jnp.where(blk == j, zrow_vec, zeros16)
                    zmsg_v[e, pl.ds(j * 16, 16)] = val

        for g in range(C // 16):
            group(g)
        pltpu.sync_copy(msg_v, acc_sh.at[dst_v], add=True)
        pltpu.sync_copy(zmsg_v, zacc_sh.at[zidx_v], add=True)
        return 0

    lax.fori_loop(0, nch, chunk, 0)

    plsc.subcore_barrier()

    for j in range((ZCH + Z8CH + NS - 1) // NS):
        ch = sid + j * NS
        @pl.when(ch < ZCH)
        def _writeout():
            pltpu.sync_copy(acc_sh.at[pl.ds(ch * ZC, ZC)],
                            wv_hbm.at[cid, pl.ds(ch * ZC, ZC)])
        @pl.when(jnp.logical_and(ch >= ZCH, ch < ZCH + Z8CH))
        def _writeout_z():
            pltpu.sync_copy(zacc_sh.at[pl.ds((ch - ZCH) * ZC, ZC)],
                            zp_hbm.at[cid, pl.ds((ch - ZCH) * ZC, ZC)])


_edge_kernel = pl.kernel(
    _edge_body,
    out_type=[
        jax.ShapeDtypeStruct((NC, N, D), jnp.float32),
        jax.ShapeDtypeStruct((NC, N8, D), jnp.float32),
    ],
    mesh=plsc.VectorSubcoreMesh(core_axis_name="c", subcore_axis_name="s",
                                num_cores=NC, num_subcores=NS),
    scratch_types=[
        pltpu.VMEM((3 * C,), jnp.int32),
        pltpu.VMEM((C,), jnp.int32),
        pltpu.VMEM((C,), jnp.int32),
        pltpu.VMEM((C, 2 * D), jnp.float32),
        pltpu.VMEM((C, D), jnp.float32),
        pltpu.VMEM((C, D), jnp.float32),
        pltpu.VMEM((C, D), jnp.float32),
        pltpu.VMEM((R * DK,), jnp.float32),
        pltpu.VMEM_SHARED((N, D), jnp.float32),
        pltpu.VMEM_SHARED((N8, D), jnp.float32),
        pltpu.SemaphoreType.DMA,
        pltpu.SemaphoreType.DMA,
    ],
)


BRT = 1000


def _ln(h, g, b):
    m = jnp.mean(h, axis=-1, keepdims=True)
    v = jnp.mean((h - m) ** 2, axis=-1, keepdims=True)
    return (h - m) / jnp.sqrt(v + 1e-5) * g + b


def _tail_body(x_ref, a_ref, zp_ref, sel_ref, wo_ref, bo_ref, g1_ref, be1_ref,
               w1_ref, b1_ref, w2_ref, b2_ref, g2_ref, be2_ref, o_ref):
    wv = a_ref[0] + a_ref[1]
    z = zp_ref[0] + zp_ref[1]
    zr = jnp.dot(z, sel_ref[...], preferred_element_type=jnp.float32)
    o = wv / zr
    h1 = x_ref[...] + jnp.dot(o, wo_ref[...],
                              preferred_element_type=jnp.float32) + bo_ref[...]
    h1 = _ln(h1, g1_ref[...], be1_ref[...])
    f = jnp.dot(h1, w1_ref[...], preferred_element_type=jnp.float32)
    f = jnp.maximum(f + b1_ref[...], 0.0)
    f = jnp.dot(f, w2_ref[...], preferred_element_type=jnp.float32) + b2_ref[...]
    o_ref[...] = _ln(h1 + f, g2_ref[...], be2_ref[...])


_tail_call = pl.pallas_call(
    _tail_body,
    grid=(N // BRT,),
    in_specs=[
        pl.BlockSpec((BRT, D), lambda i: (i, 0)),
        pl.BlockSpec((NC, BRT, D), lambda i: (0, i, 0)),
        pl.BlockSpec((NC, BRT, H), lambda i: (0, i, 0)),
        pl.BlockSpec((H, D), lambda i: (0, 0)),
        pl.BlockSpec((D, D), lambda i: (0, 0)),
        pl.BlockSpec((1, D), lambda i: (0, 0)),
        pl.BlockSpec((1, D), lambda i: (0, 0)),
        pl.BlockSpec((1, D), lambda i: (0, 0)),
        pl.BlockSpec((D, DFF), lambda i: (0, 0)),
        pl.BlockSpec((1, DFF), lambda i: (0, 0)),
        pl.BlockSpec((DFF, D), lambda i: (0, 0)),
        pl.BlockSpec((1, D), lambda i: (0, 0)),
        pl.BlockSpec((1, D), lambda i: (0, 0)),
        pl.BlockSpec((1, D), lambda i: (0, 0)),
    ],
    out_specs=pl.BlockSpec((BRT, D), lambda i: (i, 0)),
    out_shape=jax.ShapeDtypeStruct((N, D), jnp.float32),
)

_SEL = np.kron(np.eye(H, dtype=np.float32), np.ones((1, DK), np.float32))


def kernel(x, edge_index, edge_feat, rel_embed, Wq, bq, Wk, Wv, Wo, bo,
           ln1_g, ln1_b, W1, b1, W2, b2, ln2_g, ln2_b):
    Wqkv = jnp.concatenate([Wq, Wk, Wv], axis=1)
    bqkv = jnp.concatenate(
        [bq, jnp.zeros((2 * D,), jnp.float32)]).reshape(1, 3 * D)
    q, kv = _qkv_call(x, Wqkv, bqkv)

    src = edge_index[0].astype(jnp.int32)
    dst = edge_index[1].astype(jnp.int32)
    feat = edge_feat.astype(jnp.int32)
    eidx = jnp.concatenate(
        [src.reshape(TOT_CH, C), dst.reshape(TOT_CH, C),
         feat.reshape(TOT_CH, C)], axis=1)
    rel_flat = rel_embed.astype(jnp.float32).reshape(R * DK)
    wv2, zp = _edge_kernel(q, kv, rel_flat, eidx)

    z = zp.reshape(NC, N8, 8, 16)[:, : N // 8, :, :H].reshape(NC, N, H)

    sel = jnp.asarray(_SEL)
    out = _tail_call(
        x, wv2, z, sel, Wo, bo.reshape(1, D),
        ln1_g.reshape(1, D), ln1_b.reshape(1, D), W1, b1.reshape(1, DFF),
        W2, b2.reshape(1, D), ln2_g.reshape(1, D), ln2_b.reshape(1, D))
    return out


# speedup vs baseline: 1.1033x; 1.1033x over previous
import functools

import numpy as np
import jax
import jax.numpy as jnp
from jax import lax
from jax.experimental import pallas as pl
from jax.experimental.pallas import tpu as pltpu
from jax.experimental.pallas import tpu_sc as plsc

N, E, D, H, DK, R, DFF = 10000, 320000, 128, 8, 16, 100, 512
INV_SCALE = 1.0 / 4.0
NC, NS = 2, 16
NW = NC * NS
C = 32
TOT_CH = E // C
NCH_SMALL = TOT_CH // NW
WBIG = TOT_CH - NCH_SMALL * NW
NCH_BIG = NCH_SMALL + 1
ZC = 16
ZCH = N // ZC
N8 = 1280
Z8CH = N8 // ZC


BRQ = 1000


def _qkv_body(x_ref, w_ref, b_ref, q_ref, kv_ref):
    acc = jnp.dot(x_ref[...], w_ref[...], preferred_element_type=jnp.float32)
    acc = acc + b_ref[...]
    q_ref[...] = acc[:, :D]
    kv_ref[...] = acc[:, D:]


_qkv_call = pl.pallas_call(
    _qkv_body,
    grid=(N // BRQ,),
    in_specs=[
        pl.BlockSpec((BRQ, D), lambda i: (i, 0)),
        pl.BlockSpec((D, 3 * D), lambda i: (0, 0)),
        pl.BlockSpec((1, 3 * D), lambda i: (0, 0)),
    ],
    out_specs=[
        pl.BlockSpec((BRQ, D), lambda i: (i, 0)),
        pl.BlockSpec((BRQ, 2 * D), lambda i: (i, 0)),
    ],
    out_shape=[
        jax.ShapeDtypeStruct((N, D), jnp.float32),
        jax.ShapeDtypeStruct((N, 2 * D), jnp.float32),
    ],
)


def _edge_body(q_hbm, kv_hbm, rel_hbm, eidx_hbm,
               wv_hbm, zp_hbm,
               idx2, dst2, zidx2, kv2, q2, msg_v, zmsg_v, rel_v,
               acc_sh, zacc_sh, sem_kv0, sem_q0, sem_kv1, sem_q1):
    cid = lax.axis_index("c")
    sid = lax.axis_index("s")
    wid = sid * NC + cid

    zeros16 = jnp.zeros((16,), jnp.float32)

    def zrow(i, _):
        for j in range(D // 16):
            msg_v[i, pl.ds(j * 16, 16)] = zeros16
            zmsg_v[i, pl.ds(j * 16, 16)] = zeros16
        return 0
    lax.fori_loop(0, C, zrow, 0)

    pltpu.sync_copy(rel_hbm, rel_v)

    for j in range((ZCH + Z8CH + NS - 1) // NS):
        ch = sid + j * NS
        @pl.when(ch < ZCH)
        def _zero_acc():
            pltpu.sync_copy(msg_v.at[pl.ds(0, ZC)],
                            acc_sh.at[pl.ds(ch * ZC, ZC)])
        @pl.when(jnp.logical_and(ch >= ZCH, ch < ZCH + Z8CH))
        def _zero_zacc():
            pltpu.sync_copy(msg_v.at[pl.ds(0, ZC)],
                            zacc_sh.at[pl.ds((ch - ZCH) * ZC, ZC)])

    plsc.subcore_barrier()

    lane = lax.iota(jnp.int32, 16)
    perms = [jnp.bitwise_xor(lane, m)[:, None] for m in (8, 4, 2, 1)]
    bperms = [(lane * 0 + h)[:, None] for h in range(H)]
    dnums = lax.GatherDimensionNumbers(
        offset_dims=(), collapsed_slice_dims=(0,), start_index_map=(0,))

    nch = jnp.where(wid < WBIG, NCH_BIG, NCH_SMALL)
    ch0 = jnp.where(wid < WBIG, wid * NCH_BIG,
                    WBIG * NCH_BIG + (wid - WBIG) * NCH_SMALL)

    sems = ((sem_kv0, sem_q0), (sem_kv1, sem_q1))

    def fire(j, b):
        pltpu.sync_copy(eidx_hbm.at[ch0 + j], idx2.at[b])
        for t in range(C // 16):
            dvec16 = idx2[b, pl.ds(C + t * 16, 16)]
            dst2[b, pl.ds(t * 16, 16)] = dvec16
            zidx2[b, pl.ds(t * 16, 16)] = lax.shift_right_logical(dvec16, 3)
        pltpu.async_copy(kv_hbm.at[idx2.at[b].at[pl.ds(0, C)]],
                         kv2.at[pl.ds(b * C, C)], sems[b][0])
        pltpu.async_copy(q_hbm.at[idx2.at[b].at[pl.ds(C, C)]],
                         q2.at[pl.ds(b * C, C)], sems[b][1])

    def wait(b):
        pltpu.make_async_copy(kv_hbm.at[idx2.at[b].at[pl.ds(0, C)]],
                              kv2.at[pl.ds(b * C, C)], sems[b][0]).wait()
        pltpu.make_async_copy(q_hbm.at[idx2.at[b].at[pl.ds(C, C)]],
                              q2.at[pl.ds(b * C, C)], sems[b][1]).wait()

    fire(0, 0)

    def do_chunk(i, b):
        wait(b)
        @pl.when(i + 1 < nch)
        def _prefetch():
            fire(i + 1, 1 - b)

        def group(g):
            dvec = dst2[b, pl.ds(g * 16, 16)]
            fvec = idx2[b, pl.ds(2 * C + g * 16, 16)]
            for l in range(16):
                e = g * 16 + l
                f_s = fvec[l]
                d_s = dvec[l]
                ev = rel_v[pl.ds(f_s * DK, DK)]
                score_vec = zeros16
                for h in range(H):
                    kvec = kv2[b * C + e, pl.ds(h * DK, DK)]
                    qvec = q2[b * C + e, pl.ds(h * DK, DK)]
                    t = (kvec + ev) * qvec
                    for p in perms:
                        t = t + lax.gather(
                            t, p, dnums, (1,),
                            mode=lax.GatherScatterMode.PROMISE_IN_BOUNDS)
                    score_vec = jnp.where(lane == h, t, score_vec)
                escore = jnp.exp(jnp.clip(score_vec * INV_SCALE, -10.0, 10.0))
                for h in range(H):
                    sv = lax.gather(
                        escore, bperms[h], dnums, (1,),
                        mode=lax.GatherScatterMode.PROMISE_IN_BOUNDS)
                    vvec = kv2[b * C + e, pl.ds(D + h * DK, DK)]
                    msg_v[e, pl.ds(h * DK, DK)] = (vvec + ev) * sv
                zrow_vec = jnp.where(lane < H, escore, zeros16)
                blk = jnp.bitwise_and(d_s, 7)
                for j in range(8):
                    val = jnp.where(blk == j, zrow_vec, zeros16)
                    zmsg_v[e, pl.ds(j * 16, 16)] = val

        for g in range(C // 16):
            group(g)
        pltpu.sync_copy(msg_v, acc_sh.at[dst2.at[b]], add=True)
        pltpu.sync_copy(zmsg_v, zacc_sh.at[zidx2.at[b]], add=True)

    def chunk(i, _):
        parity0 = lax.rem(i, 2) == 0
        @pl.when(parity0)
        def _even():
            do_chunk(i, 0)
        @pl.when(jnp.logical_not(parity0))
        def _odd():
            do_chunk(i, 1)
        return 0

    lax.fori_loop(0, nch, chunk, 0)

    plsc.subcore_barrier()

    for j in range((ZCH + Z8CH + NS - 1) // NS):
        ch = sid + j * NS
        @pl.when(ch < ZCH)
        def _writeout():
            pltpu.sync_copy(acc_sh.at[pl.ds(ch * ZC, ZC)],
                            wv_hbm.at[cid, pl.ds(ch * ZC, ZC)])
        @pl.when(jnp.logical_and(ch >= ZCH, ch < ZCH + Z8CH))
        def _writeout_z():
            pltpu.sync_copy(zacc_sh.at[pl.ds((ch - ZCH) * ZC, ZC)],
                            zp_hbm.at[cid, pl.ds((ch - ZCH) * ZC, ZC)])


_edge_kernel = pl.kernel(
    _edge_body,
    out_type=[
        jax.ShapeDtypeStruct((NC, N, D), jnp.float32),
        jax.ShapeDtypeStruct((NC, N8, D), jnp.float32),
    ],
    mesh=plsc.VectorSubcoreMesh(core_axis_name="c", subcore_axis_name="s",
                                num_cores=NC, num_subcores=NS),
    scratch_types=[
        pltpu.VMEM((2, 3 * C), jnp.int32),
        pltpu.VMEM((2, C), jnp.int32),
        pltpu.VMEM((2, C), jnp.int32),
        pltpu.VMEM((2 * C, 2 * D), jnp.float32),
        pltpu.VMEM((2 * C, D), jnp.float32),
        pltpu.VMEM((C, D), jnp.float32),
        pltpu.VMEM((C, D), jnp.float32),
        pltpu.VMEM((R * DK,), jnp.float32),
        pltpu.VMEM_SHARED((N, D), jnp.float32),
        pltpu.VMEM_SHARED((N8, D), jnp.float32),
        pltpu.SemaphoreType.DMA,
        pltpu.SemaphoreType.DMA,
        pltpu.SemaphoreType.DMA,
        pltpu.SemaphoreType.DMA,
    ],
)


BRT = 1000


def _ln(h, g, b):
    m = jnp.mean(h, axis=-1, keepdims=True)
    v = jnp.mean((h - m) ** 2, axis=-1, keepdims=True)
    return (h - m) / jnp.sqrt(v + 1e-5) * g + b


def _tail_body(x_ref, a_ref, zp_ref, sel_ref, wo_ref, bo_ref, g1_ref, be1_ref,
               w1_ref, b1_ref, w2_ref, b2_ref, g2_ref, be2_ref, o_ref):
    wv = a_ref[0] + a_ref[1]
    z = zp_ref[0] + zp_ref[1]
    zr = jnp.dot(z, sel_ref[...], preferred_element_type=jnp.float32)
    o = wv / zr
    h1 = x_ref[...] + jnp.dot(o, wo_ref[...],
                              preferred_element_type=jnp.float32) + bo_ref[...]
    h1 = _ln(h1, g1_ref[...], be1_ref[...])
    f = jnp.dot(h1, w1_ref[...], preferred_element_type=jnp.float32)
    f = jnp.maximum(f + b1_ref[...], 0.0)
    f = jnp.dot(f, w2_ref[...], preferred_element_type=jnp.float32) + b2_ref[...]
    o_ref[...] = _ln(h1 + f, g2_ref[...], be2_ref[...])


_tail_call = pl.pallas_call(
    _tail_body,
    grid=(N // BRT,),
    in_specs=[
        pl.BlockSpec((BRT, D), lambda i: (i, 0)),
        pl.BlockSpec((NC, BRT, D), lambda i: (0, i, 0)),
        pl.BlockSpec((NC, BRT, H), lambda i: (0, i, 0)),
        pl.BlockSpec((H, D), lambda i: (0, 0)),
        pl.BlockSpec((D, D), lambda i: (0, 0)),
        pl.BlockSpec((1, D), lambda i: (0, 0)),
        pl.BlockSpec((1, D), lambda i: (0, 0)),
        pl.BlockSpec((1, D), lambda i: (0, 0)),
        pl.BlockSpec((D, DFF), lambda i: (0, 0)),
        pl.BlockSpec((1, DFF), lambda i: (0, 0)),
        pl.BlockSpec((DFF, D), lambda i: (0, 0)),
        pl.BlockSpec((1, D), lambda i: (0, 0)),
        pl.BlockSpec((1, D), lambda i: (0, 0)),
        pl.BlockSpec((1, D), lambda i: (0, 0)),
    ],
    out_specs=pl.BlockSpec((BRT, D), lambda i: (i, 0)),
    out_shape=jax.ShapeDtypeStruct((N, D), jnp.float32),
)

_SEL = np.kron(np.eye(H, dtype=np.float32), np.ones((1, DK), np.float32))


def kernel(x, edge_index, edge_feat, rel_embed, Wq, bq, Wk, Wv, Wo, bo,
           ln1_g, ln1_b, W1, b1, W2, b2, ln2_g, ln2_b):
    Wqkv = jnp.concatenate([Wq, Wk, Wv], axis=1)
    bqkv = jnp.concatenate(
        [bq, jnp.zeros((2 * D,), jnp.float32)]).reshape(1, 3 * D)
    q, kv = _qkv_call(x, Wqkv, bqkv)

    src = edge_index[0].astype(jnp.int32)
    dst = edge_index[1].astype(jnp.int32)
    feat = edge_feat.astype(jnp.int32)
    eidx = jnp.concatenate(
        [src.reshape(TOT_CH, C), dst.reshape(TOT_CH, C),
         feat.reshape(TOT_CH, C)], axis=1)
    rel_flat = rel_embed.astype(jnp.float32).reshape(R * DK)
    wv2, zp = _edge_kernel(q, kv, rel_flat, eidx)

    z = zp.reshape(NC, N8, 8, 16)[:, : N // 8, :, :H].reshape(NC, N, H)

    sel = jnp.asarray(_SEL)
    out = _tail_call(
        x, wv2, z, sel, Wo, bo.reshape(1, D),
        ln1_g.reshape(1, D), ln1_b.reshape(1, D), W1, b1.reshape(1, DFF),
        W2, b2.reshape(1, D), ln2_g.reshape(1, D), ln2_b.reshape(1, D))
    return out


# speedup vs baseline: 1.2449x; 1.1284x over previous
import functools

import numpy as np
import jax
import jax.numpy as jnp
from jax import lax
from jax.experimental import pallas as pl
from jax.experimental.pallas import tpu as pltpu
from jax.experimental.pallas import tpu_sc as plsc

N, E, D, H, DK, R, DFF = 10000, 320000, 128, 8, 16, 100, 512
INV_SCALE = 1.0 / 4.0
NC, NS = 2, 16
NW = NC * NS
C = 32
TOT_CH = E // C
NCH_SMALL = TOT_CH // NW
WBIG = TOT_CH - NCH_SMALL * NW
NCH_BIG = NCH_SMALL + 1
ZC = 16
N8 = 1280
NT = N + N8
TCH = NT // ZC


BRQ = 1000


def _qkv_body(x_ref, w_ref, b_ref, q_ref, kv_ref):
    acc = jnp.dot(x_ref[...], w_ref[...], preferred_element_type=jnp.float32)
    acc = acc + b_ref[...]
    q_ref[...] = acc[:, :D]
    kv_ref[...] = acc[:, D:]


_qkv_call = pl.pallas_call(
    _qkv_body,
    grid=(N // BRQ,),
    in_specs=[
        pl.BlockSpec((BRQ, D), lambda i: (i, 0)),
        pl.BlockSpec((D, 3 * D), lambda i: (0, 0)),
        pl.BlockSpec((1, 3 * D), lambda i: (0, 0)),
    ],
    out_specs=[
        pl.BlockSpec((BRQ, D), lambda i: (i, 0)),
        pl.BlockSpec((BRQ, 2 * D), lambda i: (i, 0)),
    ],
    out_shape=[
        jax.ShapeDtypeStruct((N, D), jnp.float32),
        jax.ShapeDtypeStruct((N, 2 * D), jnp.float32),
    ],
)


def _edge_body(q_hbm, kv_hbm, rel_hbm, eidx_hbm,
               accc_hbm,
               idx2, sidx2, kv2, q2, msgc, rel_v,
               accc_sh, sem_kv0, sem_q0, sem_kv1, sem_q1, sem_sc):
    cid = lax.axis_index("c")
    sid = lax.axis_index("s")
    wid = sid * NC + cid

    zeros16 = jnp.zeros((16,), jnp.float32)

    def zrow(i, _):
        for j in range(D // 16):
            msgc[i, pl.ds(j * 16, 16)] = zeros16
        return 0
    lax.fori_loop(0, 2 * C, zrow, 0)

    pltpu.sync_copy(rel_hbm, rel_v)

    for j in range((TCH + NS - 1) // NS):
        ch = sid + j * NS
        @pl.when(ch < TCH)
        def _zero_acc():
            pltpu.sync_copy(msgc.at[pl.ds(0, ZC)],
                            accc_sh.at[pl.ds(ch * ZC, ZC)])

    plsc.subcore_barrier()

    lane = lax.iota(jnp.int32, 16)
    perms = [jnp.bitwise_xor(lane, m)[:, None] for m in (8, 4, 2, 1)]
    bperms = [(lane * 0 + h)[:, None] for h in range(H)]
    dnums = lax.GatherDimensionNumbers(
        offset_dims=(), collapsed_slice_dims=(0,), start_index_map=(0,))

    nch = jnp.where(wid < WBIG, NCH_BIG, NCH_SMALL)
    ch0 = jnp.where(wid < WBIG, wid * NCH_BIG,
                    WBIG * NCH_BIG + (wid - WBIG) * NCH_SMALL)

    sems = ((sem_kv0, sem_q0), (sem_kv1, sem_q1))

    def fire(j, b):
        pltpu.sync_copy(eidx_hbm.at[ch0 + j], idx2.at[b])
        for t in range(C // 16):
            dvec16 = idx2[b, pl.ds(C + t * 16, 16)]
            sidx2[b, pl.ds(t * 16, 16)] = dvec16
            sidx2[b, pl.ds(C + t * 16, 16)] = (
                N + lax.shift_right_logical(dvec16, 3))
        pltpu.async_copy(kv_hbm.at[idx2.at[b].at[pl.ds(0, C)]],
                         kv2.at[pl.ds(b * C, C)], sems[b][0])
        pltpu.async_copy(q_hbm.at[idx2.at[b].at[pl.ds(C, C)]],
                         q2.at[pl.ds(b * C, C)], sems[b][1])

    def wait(b):
        pltpu.make_async_copy(kv_hbm.at[idx2.at[b].at[pl.ds(0, C)]],
                              kv2.at[pl.ds(b * C, C)], sems[b][0]).wait()
        pltpu.make_async_copy(q_hbm.at[idx2.at[b].at[pl.ds(C, C)]],
                              q2.at[pl.ds(b * C, C)], sems[b][1]).wait()

    fire(0, 0)

    def scdrain(b):
        pltpu.make_async_copy(msgc, accc_sh.at[sidx2.at[b]], sem_sc).wait()

    def do_chunk(i, b):
        wait(b)
        @pl.when(i + 1 < nch)
        def _prefetch():
            fire(i + 1, 1 - b)
        @pl.when(i > 0)
        def _drain():
            scdrain(b)

        def group(g):
            dvec = sidx2[b, pl.ds(g * 16, 16)]
            fvec = idx2[b, pl.ds(2 * C + g * 16, 16)]
            for l in range(16):
                e = g * 16 + l
                f_s = fvec[l]
                d_s = dvec[l]
                ev = rel_v[pl.ds(f_s * DK, DK)]
                score_vec = zeros16
                for h in range(H):
                    kvec = kv2[b * C + e, pl.ds(h * DK, DK)]
                    qvec = q2[b * C + e, pl.ds(h * DK, DK)]
                    t = (kvec + ev) * qvec
                    for p in perms:
                        t = t + lax.gather(
                            t, p, dnums, (1,),
                            mode=lax.GatherScatterMode.PROMISE_IN_BOUNDS)
                    score_vec = jnp.where(lane == h, t, score_vec)
                escore = jnp.exp(jnp.clip(score_vec * INV_SCALE, -10.0, 10.0))
                for h in range(H):
                    sv = lax.gather(
                        escore, bperms[h], dnums, (1,),
                        mode=lax.GatherScatterMode.PROMISE_IN_BOUNDS)
                    vvec = kv2[b * C + e, pl.ds(D + h * DK, DK)]
                    msgc[e, pl.ds(h * DK, DK)] = (vvec + ev) * sv
                zrow_vec = jnp.where(lane < H, escore, zeros16)
                blk = jnp.bitwise_and(d_s, 7)
                for j in range(8):
                    val = jnp.where(blk == j, zrow_vec, zeros16)
                    msgc[C + e, pl.ds(j * 16, 16)] = val

        for g in range(C // 16):
            group(g)
        pltpu.async_copy(msgc, accc_sh.at[sidx2.at[b]], sem_sc, add=True)

    def chunk(i, _):
        parity0 = lax.rem(i, 2) == 0
        @pl.when(parity0)
        def _even():
            do_chunk(i, 0)
        @pl.when(jnp.logical_not(parity0))
        def _odd():
            do_chunk(i, 1)
        return 0

    lax.fori_loop(0, nch, chunk, 0)

    scdrain(0)

    plsc.subcore_barrier()

    for j in range((TCH + NS - 1) // NS):
        ch = sid + j * NS
        @pl.when(ch < TCH)
        def _writeout():
            pltpu.sync_copy(accc_sh.at[pl.ds(ch * ZC, ZC)],
                            accc_hbm.at[cid, pl.ds(ch * ZC, ZC)])


_edge_kernel = pl.kernel(
    _edge_body,
    out_type=jax.ShapeDtypeStruct((NC, NT, D), jnp.float32),
    mesh=plsc.VectorSubcoreMesh(core_axis_name="c", subcore_axis_name="s",
                                num_cores=NC, num_subcores=NS),
    scratch_types=[
        pltpu.VMEM((2, 3 * C), jnp.int32),
        pltpu.VMEM((2, 2 * C), jnp.int32),
        pltpu.VMEM((2 * C, 2 * D), jnp.float32),
        pltpu.VMEM((2 * C, D), jnp.float32),
        pltpu.VMEM((2 * C, D), jnp.float32),
        pltpu.VMEM((R * DK,), jnp.float32),
        pltpu.VMEM_SHARED((NT, D), jnp.float32),
        pltpu.SemaphoreType.DMA,
        pltpu.SemaphoreType.DMA,
        pltpu.SemaphoreType.DMA,
        pltpu.SemaphoreType.DMA,
        pltpu.SemaphoreType.DMA,
    ],
)


BRT = 1000


def _ln(h, g, b):
    m = jnp.mean(h, axis=-1, keepdims=True)
    v = jnp.mean((h - m) ** 2, axis=-1, keepdims=True)
    return (h - m) / jnp.sqrt(v + 1e-5) * g + b


def _tail_body(x_ref, a_ref, zp_ref, sel_ref, wo_ref, bo_ref, g1_ref, be1_ref,
               w1_ref, b1_ref, w2_ref, b2_ref, g2_ref, be2_ref, o_ref):
    wv = a_ref[0] + a_ref[1]
    z = zp_ref[0] + zp_ref[1]
    zr = jnp.dot(z, sel_ref[...], preferred_element_type=jnp.float32)
    o = wv / zr
    h1 = x_ref[...] + jnp.dot(o, wo_ref[...],
                              preferred_element_type=jnp.float32) + bo_ref[...]
    h1 = _ln(h1, g1_ref[...], be1_ref[...])
    f = jnp.dot(h1, w1_ref[...], preferred_element_type=jnp.float32)
    f = jnp.maximum(f + b1_ref[...], 0.0)
    f = jnp.dot(f, w2_ref[...], preferred_element_type=jnp.float32) + b2_ref[...]
    o_ref[...] = _ln(h1 + f, g2_ref[...], be2_ref[...])


_tail_call = pl.pallas_call(
    _tail_body,
    grid=(N // BRT,),
    in_specs=[
        pl.BlockSpec((BRT, D), lambda i: (i, 0)),
        pl.BlockSpec((NC, BRT, D), lambda i: (0, i, 0)),
        pl.BlockSpec((NC, BRT, H), lambda i: (0, i, 0)),
        pl.BlockSpec((H, D), lambda i: (0, 0)),
        pl.BlockSpec((D, D), lambda i: (0, 0)),
        pl.BlockSpec((1, D), lambda i: (0, 0)),
        pl.BlockSpec((1, D), lambda i: (0, 0)),
        pl.BlockSpec((1, D), lambda i: (0, 0)),
        pl.BlockSpec((D, DFF), lambda i: (0, 0)),
        pl.BlockSpec((1, DFF), lambda i: (0, 0)),
        pl.BlockSpec((DFF, D), lambda i: (0, 0)),
        pl.BlockSpec((1, D), lambda i: (0, 0)),
        pl.BlockSpec((1, D), lambda i: (0, 0)),
        pl.BlockSpec((1, D), lambda i: (0, 0)),
    ],
    out_specs=pl.BlockSpec((BRT, D), lambda i: (i, 0)),
    out_shape=jax.ShapeDtypeStruct((N, D), jnp.float32),
)

_SEL = np.kron(np.eye(H, dtype=np.float32), np.ones((1, DK), np.float32))


def kernel(x, edge_index, edge_feat, rel_embed, Wq, bq, Wk, Wv, Wo, bo,
           ln1_g, ln1_b, W1, b1, W2, b2, ln2_g, ln2_b):
    Wqkv = jnp.concatenate([Wq, Wk, Wv], axis=1)
    bqkv = jnp.concatenate(
        [bq, jnp.zeros((2 * D,), jnp.float32)]).reshape(1, 3 * D)
    q, kv = _qkv_call(x, Wqkv, bqkv)

    src = edge_index[0].astype(jnp.int32)
    dst = edge_index[1].astype(jnp.int32)
    feat = edge_feat.astype(jnp.int32)
    eidx = jnp.concatenate(
        [src.reshape(TOT_CH, C), dst.reshape(TOT_CH, C),
         feat.reshape(TOT_CH, C)], axis=1)
    rel_flat = rel_embed.astype(jnp.float32).reshape(R * DK)
    acc = _edge_kernel(q, kv, rel_flat, eidx)
    wv2 = acc[:, :N, :]
    zp = acc[:, N:, :]

    z = zp.reshape(NC, N8, 8, 16)[:, : N // 8, :, :H].reshape(NC, N, H)

    sel = jnp.asarray(_SEL)
    out = _tail_call(
        x, wv2, z, sel, Wo, bo.reshape(1, D),
        ln1_g.reshape(1, D), ln1_b.reshape(1, D), W1, b1.reshape(1, DFF),
        W2, b2.reshape(1, D), ln2_g.reshape(1, D), ln2_b.reshape(1, D))
    return out


# speedup vs baseline: 1.2539x; 1.0072x over previous
import functools

import numpy as np
import jax
import jax.numpy as jnp
from jax import lax
from jax.experimental import pallas as pl
from jax.experimental.pallas import tpu as pltpu
from jax.experimental.pallas import tpu_sc as plsc

N, E, D, H, DK, R, DFF = 10000, 320000, 128, 8, 16, 100, 512
INV_SCALE = 1.0 / 4.0
NC, NS = 2, 16
NW = NC * NS
C = 32
TOT_CH = E // C
NCH_SMALL = TOT_CH // NW
WBIG = TOT_CH - NCH_SMALL * NW
NCH_BIG = NCH_SMALL + 1
ZC = 16
N8 = 1280
NT = N + N8
TCH = NT // ZC


BRQ = 1000


def _qkv_body(x_ref, w_ref, b_ref, q_ref, kv_ref):
    acc = jnp.dot(x_ref[...], w_ref[...], preferred_element_type=jnp.float32)
    acc = acc + b_ref[...]
    q_ref[...] = acc[:, :D]
    kv_ref[...] = acc[:, D:]


_qkv_call = pl.pallas_call(
    _qkv_body,
    grid=(N // BRQ,),
    in_specs=[
        pl.BlockSpec((BRQ, D), lambda i: (i, 0)),
        pl.BlockSpec((D, 3 * D), lambda i: (0, 0)),
        pl.BlockSpec((1, 3 * D), lambda i: (0, 0)),
    ],
    out_specs=[
        pl.BlockSpec((BRQ, D), lambda i: (i, 0)),
        pl.BlockSpec((BRQ, 2 * D), lambda i: (i, 0)),
    ],
    out_shape=[
        jax.ShapeDtypeStruct((N, D), jnp.float32),
        jax.ShapeDtypeStruct((N, 2 * D), jnp.float32),
    ],
)


def _edge_body(q_hbm, kv_hbm, rel_hbm, eidx_hbm,
               accc_hbm,
               idx2, sidx2, feat2, ssidx, kv2, q2, msgc, rel_v,
               accc_sh, sem_kv0, sem_q0, sem_kv1, sem_q1, sem_sc,
               sem_i0, sem_i1):
    cid = lax.axis_index("c")
    sid = lax.axis_index("s")
    wid = sid * NC + cid

    zeros16 = jnp.zeros((16,), jnp.float32)

    def zrow(i, _):
        for j in range(D // 16):
            msgc[i, pl.ds(j * 16, 16)] = zeros16
        return 0
    lax.fori_loop(0, 2 * C, zrow, 0)

    pltpu.sync_copy(rel_hbm, rel_v)

    for j in range((TCH + NS - 1) // NS):
        ch = sid + j * NS
        @pl.when(ch < TCH)
        def _zero_acc():
            pltpu.sync_copy(msgc.at[pl.ds(0, ZC)],
                            accc_sh.at[pl.ds(ch * ZC, ZC)])

    plsc.subcore_barrier()

    lane = lax.iota(jnp.int32, 16)
    perms = [jnp.bitwise_xor(lane, m)[:, None] for m in (8, 4, 2, 1)]
    bperms = [(lane * 0 + h)[:, None] for h in range(H)]
    dnums = lax.GatherDimensionNumbers(
        offset_dims=(), collapsed_slice_dims=(0,), start_index_map=(0,))

    nch = jnp.where(wid < WBIG, NCH_BIG, NCH_SMALL)
    ch0 = jnp.where(wid < WBIG, wid * NCH_BIG,
                    WBIG * NCH_BIG + (wid - WBIG) * NCH_SMALL)

    sems = ((sem_kv0, sem_q0), (sem_kv1, sem_q1))
    isems = (sem_i0, sem_i1)

    def fire_idx(j, b):
        pltpu.async_copy(eidx_hbm.at[ch0 + j], idx2.at[b], isems[b])

    def drain_idx(b):
        pltpu.make_async_copy(eidx_hbm.at[ch0], idx2.at[b], isems[b]).wait()

    def extract(b):
        for t in range(C // 16):
            dvec16 = idx2[b, pl.ds(C + t * 16, 16)]
            sidx2[b, pl.ds(t * 16, 16)] = dvec16
            sidx2[b, pl.ds(C + t * 16, 16)] = (
                N + lax.shift_right_logical(dvec16, 3))
            feat2[b, pl.ds(t * 16, 16)] = idx2[b, pl.ds(2 * C + t * 16, 16)]

    def fire_gathers(b):
        pltpu.async_copy(kv_hbm.at[idx2.at[b].at[pl.ds(0, C)]],
                         kv2.at[pl.ds(b * C, C)], sems[b][0])
        pltpu.async_copy(q_hbm.at[idx2.at[b].at[pl.ds(C, C)]],
                         q2.at[pl.ds(b * C, C)], sems[b][1])

    def wait(b):
        pltpu.make_async_copy(kv_hbm.at[idx2.at[b].at[pl.ds(0, C)]],
                              kv2.at[pl.ds(b * C, C)], sems[b][0]).wait()
        pltpu.make_async_copy(q_hbm.at[idx2.at[b].at[pl.ds(C, C)]],
                              q2.at[pl.ds(b * C, C)], sems[b][1]).wait()

    fire_idx(0, 0)
    drain_idx(0)
    extract(0)
    fire_gathers(0)
    @pl.when(nch > 1)
    def _pro1():
        fire_idx(1, 1)

    def scdrain():
        pltpu.make_async_copy(msgc, accc_sh.at[ssidx], sem_sc).wait()

    def do_chunk(i, b):
        nb = 1 - b
        @pl.when(i + 1 < nch)
        def _prefetch():
            drain_idx(nb)
            extract(nb)
            fire_gathers(nb)
        @pl.when(i + 2 < nch)
        def _prefetch_idx():
            fire_idx(i + 2, b)
        wait(b)
        @pl.when(i > 0)
        def _drain():
            scdrain()

        def group(g):
            dvec = sidx2[b, pl.ds(g * 16, 16)]
            fvec = feat2[b, pl.ds(g * 16, 16)]
            for l in range(16):
                e = g * 16 + l
                f_s = fvec[l]
                d_s = dvec[l]
                ev = rel_v[pl.ds(f_s * DK, DK)]
                score_vec = zeros16
                for h in range(H):
                    kvec = kv2[b * C + e, pl.ds(h * DK, DK)]
                    qvec = q2[b * C + e, pl.ds(h * DK, DK)]
                    t = (kvec + ev) * qvec
                    for p in perms:
                        t = t + lax.gather(
                            t, p, dnums, (1,),
                            mode=lax.GatherScatterMode.PROMISE_IN_BOUNDS)
                    score_vec = jnp.where(lane == h, t, score_vec)
                escore = jnp.exp(jnp.clip(score_vec * INV_SCALE, -10.0, 10.0))
                for h in range(H):
                    sv = lax.gather(
                        escore, bperms[h], dnums, (1,),
                        mode=lax.GatherScatterMode.PROMISE_IN_BOUNDS)
                    vvec = kv2[b * C + e, pl.ds(D + h * DK, DK)]
                    msgc[e, pl.ds(h * DK, DK)] = (vvec + ev) * sv
                zrow_vec = jnp.where(lane < H, escore, zeros16)
                blk = jnp.bitwise_and(d_s, 7)
                for j in range(8):
                    val = jnp.where(blk == j, zrow_vec, zeros16)
                    msgc[C + e, pl.ds(j * 16, 16)] = val

        for g in range(C // 16):
            group(g)
        for t in range((2 * C) // 16):
            ssidx[pl.ds(t * 16, 16)] = sidx2[b, pl.ds(t * 16, 16)]
        pltpu.async_copy(msgc, accc_sh.at[ssidx], sem_sc, add=True)

    def chunk(i, _):
        parity0 = lax.rem(i, 2) == 0
        @pl.when(parity0)
        def _even():
            do_chunk(i, 0)
        @pl.when(jnp.logical_not(parity0))
        def _odd():
            do_chunk(i, 1)
        return 0

    lax.fori_loop(0, nch, chunk, 0)

    scdrain()

    plsc.subcore_barrier()

    for j in range((TCH + NS - 1) // NS):
        ch = sid + j * NS
        @pl.when(ch < TCH)
        def _writeout():
            pltpu.sync_copy(accc_sh.at[pl.ds(ch * ZC, ZC)],
                            accc_hbm.at[cid, pl.ds(ch * ZC, ZC)])


_edge_kernel = pl.kernel(
    _edge_body,
    out_type=jax.ShapeDtypeStruct((NC, NT, D), jnp.float32),
    mesh=plsc.VectorSubcoreMesh(core_axis_name="c", subcore_axis_name="s",
                                num_cores=NC, num_subcores=NS),
    scratch_types=[
        pltpu.VMEM((2, 3 * C), jnp.int32),
        pltpu.VMEM((2, 2 * C), jnp.int32),
        pltpu.VMEM((2, C), jnp.int32),
        pltpu.VMEM((2 * C,), jnp.int32),
        pltpu.VMEM((2 * C, 2 * D), jnp.float32),
        pltpu.VMEM((2 * C, D), jnp.float32),
        pltpu.VMEM((2 * C, D), jnp.float32),
        pltpu.VMEM((R * DK,), jnp.float32),
        pltpu.VMEM_SHARED((NT, D), jnp.float32),
        pltpu.SemaphoreType.DMA,
        pltpu.SemaphoreType.DMA,
        pltpu.SemaphoreType.DMA,
        pltpu.SemaphoreType.DMA,
        pltpu.SemaphoreType.DMA,
        pltpu.SemaphoreType.DMA,
        pltpu.SemaphoreType.DMA,
    ],
)


BRT = 1000


def _ln(h, g, b):
    m = jnp.mean(h, axis=-1, keepdims=True)
    v = jnp.mean((h - m) ** 2, axis=-1, keepdims=True)
    return (h - m) / jnp.sqrt(v + 1e-5) * g + b


def _tail_body(x_ref, a_ref, zp_ref, sel_ref, wo_ref, bo_ref, g1_ref, be1_ref,
               w1_ref, b1_ref, w2_ref, b2_ref, g2_ref, be2_ref, o_ref):
    wv = a_ref[0] + a_ref[1]
    z = zp_ref[0] + zp_ref[1]
    zr = jnp.dot(z, sel_ref[...], preferred_element_type=jnp.float32)
    o = wv / zr
    h1 = x_ref[...] + jnp.dot(o, wo_ref[...],
                              preferred_element_type=jnp.float32) + bo_ref[...]
    h1 = _ln(h1, g1_ref[...], be1_ref[...])
    f = jnp.dot(h1, w1_ref[...], preferred_element_type=jnp.float32)
    f = jnp.maximum(f + b1_ref[...], 0.0)
    f = jnp.dot(f, w2_ref[...], preferred_element_type=jnp.float32) + b2_ref[...]
    o_ref[...] = _ln(h1 + f, g2_ref[...], be2_ref[...])


_tail_call = pl.pallas_call(
    _tail_body,
    grid=(N // BRT,),
    in_specs=[
        pl.BlockSpec((BRT, D), lambda i: (i, 0)),
        pl.BlockSpec((NC, BRT, D), lambda i: (0, i, 0)),
        pl.BlockSpec((NC, BRT, H), lambda i: (0, i, 0)),
        pl.BlockSpec((H, D), lambda i: (0, 0)),
        pl.BlockSpec((D, D), lambda i: (0, 0)),
        pl.BlockSpec((1, D), lambda i: (0, 0)),
        pl.BlockSpec((1, D), lambda i: (0, 0)),
        pl.BlockSpec((1, D), lambda i: (0, 0)),
        pl.BlockSpec((D, DFF), lambda i: (0, 0)),
        pl.BlockSpec((1, DFF), lambda i: (0, 0)),
        pl.BlockSpec((DFF, D), lambda i: (0, 0)),
        pl.BlockSpec((1, D), lambda i: (0, 0)),
        pl.BlockSpec((1, D), lambda i: (0, 0)),
        pl.BlockSpec((1, D), lambda i: (0, 0)),
    ],
    out_specs=pl.BlockSpec((BRT, D), lambda i: (i, 0)),
    out_shape=jax.ShapeDtypeStruct((N, D), jnp.float32),
)

_SEL = np.kron(np.eye(H, dtype=np.float32), np.ones((1, DK), np.float32))


def kernel(x, edge_index, edge_feat, rel_embed, Wq, bq, Wk, Wv, Wo, bo,
           ln1_g, ln1_b, W1, b1, W2, b2, ln2_g, ln2_b):
    Wqkv = jnp.concatenate([Wq, Wk, Wv], axis=1)
    bqkv = jnp.concatenate(
        [bq, jnp.zeros((2 * D,), jnp.float32)]).reshape(1, 3 * D)
    q, kv = _qkv_call(x, Wqkv, bqkv)

    src = edge_index[0].astype(jnp.int32)
    dst = edge_index[1].astype(jnp.int32)
    feat = edge_feat.astype(jnp.int32)
    eidx = jnp.concatenate(
        [src.reshape(TOT_CH, C), dst.reshape(TOT_CH, C),
         feat.reshape(TOT_CH, C)], axis=1)
    rel_flat = rel_embed.astype(jnp.float32).reshape(R * DK)
    acc = _edge_kernel(q, kv, rel_flat, eidx)
    wv2 = acc[:, :N, :]
    zp = acc[:, N:, :]

    z = zp.reshape(NC, N8, 8, 16)[:, : N // 8, :, :H].reshape(NC, N, H)

    sel = jnp.asarray(_SEL)
    out = _tail_call(
        x, wv2, z, sel, Wo, bo.reshape(1, D),
        ln1_g.reshape(1, D), ln1_b.reshape(1, D), W1, b1.reshape(1, DFF),
        W2, b2.reshape(1, D), ln2_g.reshape(1, D), ln2_b.reshape(1, D))
    return out
